# baseline (speedup 1.0000x reference)
import functools
import math

import jax
import jax.numpy as jnp
from jax import lax
from jax.experimental import pallas as pl
from jax.experimental.pallas import tpu as pltpu
from jax.experimental.pallas import tpu_sc as plsc

D_MODEL = 64
_SCALE = math.sqrt(D_MODEL)

_INFO = plsc.get_sparse_core_info()
_NC, _NS, _L = _INFO.num_cores, _INFO.num_subcores, _INFO.num_lanes
_NW = _NC * _NS


def _make_kernel(B, D, C):
    assert B % (_NW * C) == 0
    b_per_w = B // _NW
    n_chunks = b_per_w // C
    vecs_per_row = D // _L

    mesh = plsc.VectorSubcoreMesh(core_axis_name="c", subcore_axis_name="s")

    @functools.partial(
        pl.kernel,
        out_type=jax.ShapeDtypeStruct((B, D), jnp.float32),
        mesh=mesh,
        scratch_types=[
            pltpu.VMEM((C,), jnp.int32),
            pltpu.VMEM((C, D), jnp.float32),
            pltpu.SemaphoreType.DMA,
        ],
        compiler_params=pltpu.CompilerParams(use_tc_tiling_on_sc=False),
    )
    def k(idx_hbm, lut_hbm, out_hbm, idx_v, rows_v, sem):
        wid = lax.axis_index("s") * _NC + lax.axis_index("c")
        base = wid * b_per_w

        def chunk_body(i, _):
            off = base + i * C
            pltpu.sync_copy(idx_hbm.at[pl.ds(off, C)], idx_v)
            pltpu.async_copy(lut_hbm.at[idx_v], rows_v, sem).wait()

            def row_body(r, _):
                for v in range(vecs_per_row):
                    sl = pl.ds(v * _L, _L)
                    rows_v[r, sl] = rows_v[r, sl] * _SCALE
                return 0

            lax.fori_loop(0, C, row_body, 0, unroll=2)
            pltpu.sync_copy(rows_v, out_hbm.at[pl.ds(off, C)])
            return 0

        lax.fori_loop(0, n_chunks, chunk_body, 0)

    return k


def kernel(x, lut):
    B = x.shape[0] * x.shape[1]
    xf = x.reshape(B).astype(jnp.int32)
    out = _make_kernel(B, D_MODEL, 512)(xf, lut)
    return out.reshape(x.shape[0], x.shape[1], D_MODEL)


# speedup vs baseline: 1.0861x; 1.0861x over previous
import functools
import math

import jax
import jax.numpy as jnp
from jax import lax
from jax.experimental import pallas as pl
from jax.experimental.pallas import tpu as pltpu
from jax.experimental.pallas import tpu_sc as plsc

D_MODEL = 64
_SCALE = math.sqrt(D_MODEL)

_INFO = plsc.get_sparse_core_info()
_NC, _NS, _L = _INFO.num_cores, _INFO.num_subcores, _INFO.num_lanes
_NW = _NC * _NS


def _make_kernel(B, D, C, n_buf):
    assert B % (_NW * C * n_buf) == 0
    b_per_w = B // _NW
    n_chunks = b_per_w // C
    n_groups = n_chunks // n_buf
    vecs_per_row = D // _L

    mesh = plsc.VectorSubcoreMesh(core_axis_name="c", subcore_axis_name="s")

    @functools.partial(
        pl.kernel,
        out_type=jax.ShapeDtypeStruct((B, D), jnp.float32),
        mesh=mesh,
        scratch_types=(
            [pltpu.VMEM((b_per_w,), jnp.int32)]
            + [pltpu.VMEM((C, D), jnp.float32) for _ in range(n_buf)]
            + [pltpu.SemaphoreType.DMA for _ in range(2 * n_buf)]
        ),
        compiler_params=pltpu.CompilerParams(use_tc_tiling_on_sc=False),
    )
    def k(idx_hbm, lut_hbm, out_hbm, idx_all, *bufs_and_sems):
        rows = list(bufs_and_sems[:n_buf])
        sin = list(bufs_and_sems[n_buf : 2 * n_buf])
        sout = list(bufs_and_sems[2 * n_buf : 3 * n_buf])

        wid = lax.axis_index("s") * _NC + lax.axis_index("c")
        base = wid * b_per_w

        pltpu.sync_copy(idx_hbm.at[pl.ds(base, b_per_w)], idx_all)

        def fire_gather(c, b):
            pltpu.async_copy(
                lut_hbm.at[idx_all.at[pl.ds(c * C, C)]], rows[b], sin[b]
            )

        def wait_gather(b):
            pltpu.make_async_copy(
                lut_hbm.at[idx_all.at[pl.ds(0, C)]], rows[b], sin[b]
            ).wait()

        def fire_out(c, b):
            pltpu.async_copy(rows[b], out_hbm.at[pl.ds(base + c * C, C)], sout[b])

        def wait_out(b):
            pltpu.make_async_copy(
                rows[b], out_hbm.at[pl.ds(base, C)], sout[b]
            ).wait()

        for b in range(n_buf - 1):
            fire_gather(b, b)

        def group_body(g, _):
            for b in range(n_buf):
                c = g * n_buf + b
                bp = (b - 1) % n_buf
                t = c + n_buf - 1

                @pl.when(jnp.logical_and(t < n_chunks, c >= 1))
                def _():
                    wait_out(bp)

                @pl.when(t < n_chunks)
                def _():
                    fire_gather(t, bp)

                wait_gather(b)

                def row_body(r, _):
                    for v in range(vecs_per_row):
                        sl = pl.ds(v * _L, _L)
                        rows[b][r, sl] = rows[b][r, sl] * _SCALE
                    return 0

                lax.fori_loop(0, C, row_body, 0, unroll=4)
                fire_out(c, b)
            return 0

        lax.fori_loop(0, n_groups, group_body, 0)

        for b in range(n_buf):
            wait_out(b)

    return k


def kernel(x, lut):
    B = x.shape[0] * x.shape[1]
    xf = x.reshape(B).astype(jnp.int32)
    out = _make_kernel(B, D_MODEL, 256, 4)(xf, lut)
    return out.reshape(x.shape[0], x.shape[1], D_MODEL)
